# baseline (speedup 1.0000x reference)
import jax
import jax.numpy as jnp
from jax.experimental import pallas as pl


def kernel(x, angles, plane_i, plane_j):
    B, S, D = x.shape
    T = B * S
    NP = angles.shape[0]
    P2 = 2 * NP

    cos = jnp.cos(angles).astype(jnp.float32)
    sin = jnp.sin(angles).astype(jnp.float32)
    pi = plane_i.astype(jnp.int32)
    pj = plane_j.astype(jnp.int32)

    selfc = jnp.concatenate([pi, pj])
    partc = jnp.concatenate([pj, pi])
    As = jax.nn.one_hot(selfc, D, dtype=jnp.float32).T
    Ap = jax.nn.one_hot(partc, D, dtype=jnp.float32).T
    Bm = jax.nn.one_hot(selfc, D, dtype=jnp.float32)
    ws = jnp.concatenate([cos - 1.0, cos - 1.0]).reshape(1, P2)
    wp = jnp.concatenate([-sin, sin]).reshape(1, P2)

    xf = x.reshape(T, D)
    BLK = 1024
    grid = (T // BLK,)

    def body(x_ref, as_ref, ap_ref, b_ref, ws_ref, wp_ref, o_ref):
        xb = x_ref[...]
        xb16 = xb.astype(jnp.bfloat16)
        a2 = jnp.concatenate([as_ref[...], ap_ref[...]], axis=1)
        z2 = jnp.dot(xb16, a2.astype(jnp.bfloat16),
                     preferred_element_type=jnp.float32)
        dv = z2[:, :P2] * ws_ref[...] + z2[:, P2:] * wp_ref[...]
        delta = jnp.dot(dv.astype(jnp.bfloat16),
                        b_ref[...].astype(jnp.bfloat16),
                        preferred_element_type=jnp.float32)
        o_ref[...] = xb + delta

    out = pl.pallas_call(
        body,
        grid=grid,
        in_specs=[
            pl.BlockSpec((BLK, D), lambda i: (i, 0)),
            pl.BlockSpec((D, P2), lambda i: (0, 0)),
            pl.BlockSpec((D, P2), lambda i: (0, 0)),
            pl.BlockSpec((P2, D), lambda i: (0, 0)),
            pl.BlockSpec((1, P2), lambda i: (0, 0)),
            pl.BlockSpec((1, P2), lambda i: (0, 0)),
        ],
        out_specs=pl.BlockSpec((BLK, D), lambda i: (i, 0)),
        out_shape=jax.ShapeDtypeStruct((T, D), jnp.float32),
    )(xf, As, Ap, Bm, ws, wp)
    return out.reshape(B, S, D)


# speedup vs baseline: 1.0007x; 1.0007x over previous
import jax
import jax.numpy as jnp
from jax.experimental import pallas as pl


def kernel(x, angles, plane_i, plane_j):
    B, S, D = x.shape
    T = B * S
    NP = angles.shape[0]
    P2 = 2 * NP

    cos = jnp.cos(angles).astype(jnp.float32)
    sin = jnp.sin(angles).astype(jnp.float32)
    pi = plane_i.astype(jnp.int32)
    pj = plane_j.astype(jnp.int32)

    selfc = jnp.concatenate([pi, pj])
    partc = jnp.concatenate([pj, pi])
    A2 = jax.nn.one_hot(jnp.concatenate([selfc, partc]), D,
                        dtype=jnp.float32).T
    Bm = jax.nn.one_hot(selfc, D, dtype=jnp.float32)
    ws = jnp.concatenate([cos - 1.0, cos - 1.0]).reshape(1, P2)
    wp = jnp.concatenate([-sin, sin]).reshape(1, P2)

    xf = x.reshape(T, D)
    BLK = 1024
    grid = (T // BLK,)

    def body(x_ref, a2_ref, b_ref, ws_ref, wp_ref, o_ref):
        xb = x_ref[...]
        z2 = jnp.dot(xb.astype(jnp.bfloat16),
                     a2_ref[...].astype(jnp.bfloat16),
                     preferred_element_type=jnp.float32)
        dv = z2[:, :P2] * ws_ref[...] + z2[:, P2:] * wp_ref[...]
        delta = jnp.dot(dv.astype(jnp.bfloat16),
                        b_ref[...].astype(jnp.bfloat16),
                        preferred_element_type=jnp.float32)
        o_ref[...] = xb + delta

    out = pl.pallas_call(
        body,
        grid=grid,
        in_specs=[
            pl.BlockSpec((BLK, D), lambda i: (i, 0)),
            pl.BlockSpec((D, 2 * P2), lambda i: (0, 0)),
            pl.BlockSpec((P2, D), lambda i: (0, 0)),
            pl.BlockSpec((1, P2), lambda i: (0, 0)),
            pl.BlockSpec((1, P2), lambda i: (0, 0)),
        ],
        out_specs=pl.BlockSpec((BLK, D), lambda i: (i, 0)),
        out_shape=jax.ShapeDtypeStruct((T, D), jnp.float32),
    )(xf, A2, Bm, ws, wp)
    return out.reshape(B, S, D)


# speedup vs baseline: 1.0449x; 1.0441x over previous
import jax
import jax.numpy as jnp
from jax.experimental import pallas as pl


def kernel(x, angles, plane_i, plane_j):
    B, S, D = x.shape
    T = B * S
    NP = angles.shape[0]
    P2 = 2 * NP

    cos = jnp.cos(angles).astype(jnp.float32)
    sin = jnp.sin(angles).astype(jnp.float32)
    pi = plane_i.astype(jnp.int32)
    pj = plane_j.astype(jnp.int32)

    selfc = jnp.concatenate([pi, pj])
    partc = jnp.concatenate([pj, pi])
    As = jax.nn.one_hot(selfc, D, dtype=jnp.float32).T
    Ap = jax.nn.one_hot(partc, D, dtype=jnp.float32).T
    Bm = jax.nn.one_hot(selfc, D, dtype=jnp.float32)
    ws = jnp.concatenate([cos - 1.0, cos - 1.0]).reshape(1, P2)
    wp = jnp.concatenate([-sin, sin]).reshape(1, P2)

    xf = x.reshape(T, D)
    BLK = 1024
    grid = (T // BLK,)

    def body(x_ref, as_ref, ap_ref, b_ref, ws_ref, wp_ref, o_ref):
        xb = x_ref[...]
        xb16 = xb.astype(jnp.bfloat16)
        zs = jnp.dot(xb16, as_ref[...].astype(jnp.bfloat16),
                     preferred_element_type=jnp.float32)
        zp = jnp.dot(xb16, ap_ref[...].astype(jnp.bfloat16),
                     preferred_element_type=jnp.float32)
        dv = zs * ws_ref[...] + zp * wp_ref[...]
        delta = jnp.dot(dv.astype(jnp.bfloat16),
                        b_ref[...].astype(jnp.bfloat16),
                        preferred_element_type=jnp.float32)
        o_ref[...] = xb + delta

    out = pl.pallas_call(
        body,
        grid=grid,
        in_specs=[
            pl.BlockSpec((BLK, D), lambda i: (i, 0)),
            pl.BlockSpec((D, P2), lambda i: (0, 0)),
            pl.BlockSpec((D, P2), lambda i: (0, 0)),
            pl.BlockSpec((P2, D), lambda i: (0, 0)),
            pl.BlockSpec((1, P2), lambda i: (0, 0)),
            pl.BlockSpec((1, P2), lambda i: (0, 0)),
        ],
        out_specs=pl.BlockSpec((BLK, D), lambda i: (i, 0)),
        out_shape=jax.ShapeDtypeStruct((T, D), jnp.float32),
    )(xf, As, Ap, Bm, ws, wp)
    return out.reshape(B, S, D)
